# baseline (speedup 1.0000x reference)
import jax
import jax.numpy as jnp
from jax.experimental import pallas as pl

_H = 16
_D = 64
_K = 1024
_CHUNK = 512


def _vq_kernel(x_ref, m_ref, o_ref):
    mt = m_ref[0]
    m_sq = jnp.sum(mt * mt, axis=0, keepdims=True)
    xc = x_ref[0]
    s = jax.lax.dot_general(
        xc, mt, (((1,), (0,)), ((), ())),
        preferred_element_type=jnp.float32)
    d = m_sq - 2.0 * s
    o_ref[0, 0, 0, :] = jnp.argmin(d, axis=1).astype(jnp.int32)


def kernel(x, means):
    b, n, feat = x.shape
    bn = b * n
    xh = x.reshape(bn, _H, _D).transpose(1, 0, 2)
    mt = means.transpose(0, 2, 1)
    nc = bn // _CHUNK
    out = pl.pallas_call(
        _vq_kernel,
        grid=(_H, nc),
        in_specs=[
            pl.BlockSpec((1, _CHUNK, _D), lambda h, c: (h, c, 0)),
            pl.BlockSpec((1, _D, _K), lambda h, c: (h, 0, 0)),
        ],
        out_specs=pl.BlockSpec((1, 1, 1, _CHUNK), lambda h, c: (h, c, 0, 0)),
        out_shape=jax.ShapeDtypeStruct((_H, nc, 1, _CHUNK), jnp.int32),
    )(xh, mt)
    return out.reshape(_H, bn).T.reshape(b, n, _H)


# speedup vs baseline: 1.0494x; 1.0494x over previous
import jax
import jax.numpy as jnp
from jax.experimental import pallas as pl

_H = 16
_D = 64
_K = 1024
_DA = 72
_CHUNK = 512


def _vq_kernel(a_ref, x_ref, o_ref):
    a = a_ref[0]
    xc = x_ref[0]
    d = jax.lax.dot_general(
        a, xc, (((1,), (0,)), ((), ())),
        preferred_element_type=jnp.float32)
    o_ref[0, 0, 0, :] = jnp.argmin(d, axis=0).astype(jnp.int32)


def kernel(x, means):
    b, n, feat = x.shape
    bn = b * n
    h, k, dim = means.shape
    m_sq = jnp.sum(means * means, axis=-1, keepdims=True)
    a = jnp.concatenate(
        [-2.0 * means, m_sq,
         jnp.zeros((h, k, _DA - dim - 1), jnp.float32)], axis=-1)
    xt = x.reshape(bn, h, dim).transpose(1, 2, 0)
    xa = jnp.concatenate(
        [xt, jnp.ones((h, 1, bn), jnp.float32),
         jnp.zeros((h, _DA - dim - 1, bn), jnp.float32)], axis=1)
    nc = bn // _CHUNK
    out = pl.pallas_call(
        _vq_kernel,
        grid=(_H, nc),
        in_specs=[
            pl.BlockSpec((1, _K, _DA), lambda hh, c: (hh, 0, 0)),
            pl.BlockSpec((1, _DA, _CHUNK), lambda hh, c: (hh, 0, c)),
        ],
        out_specs=pl.BlockSpec((1, 1, 1, _CHUNK), lambda hh, c: (hh, c, 0, 0)),
        out_shape=jax.ShapeDtypeStruct((_H, nc, 1, _CHUNK), jnp.int32),
    )(a, xa)
    return out.reshape(_H, bn).T.reshape(b, n, _H)


# speedup vs baseline: 1.1247x; 1.0717x over previous
import jax
import jax.numpy as jnp
from jax.experimental import pallas as pl

_H = 16
_D = 64
_K = 1024
_CHUNK = 512


def _vq_kernel(a_ref, msq_ref, x_ref, o_ref):
    a = a_ref[0]
    xc = x_ref[0]
    s = jax.lax.dot_general(
        a, xc, (((1,), (0,)), ((), ())),
        preferred_element_type=jnp.float32)
    d = s + msq_ref[0]
    o_ref[0, 0, 0, :] = jnp.argmin(d, axis=0).astype(jnp.int32)


def kernel(x, means):
    b, n, feat = x.shape
    bn = b * n
    h, k, dim = means.shape
    a = -2.0 * means
    m_sq = jnp.sum(means * means, axis=-1, keepdims=True)
    xt = x.reshape(bn, h, dim).transpose(1, 2, 0)
    nc = bn // _CHUNK
    out = pl.pallas_call(
        _vq_kernel,
        grid=(_H, nc),
        in_specs=[
            pl.BlockSpec((1, _K, _D), lambda hh, c: (hh, 0, 0)),
            pl.BlockSpec((1, _K, 1), lambda hh, c: (hh, 0, 0)),
            pl.BlockSpec((1, _D, _CHUNK), lambda hh, c: (hh, 0, c)),
        ],
        out_specs=pl.BlockSpec((1, 1, 1, _CHUNK), lambda hh, c: (hh, c, 0, 0)),
        out_shape=jax.ShapeDtypeStruct((_H, nc, 1, _CHUNK), jnp.int32),
    )(a, m_sq, xt)
    return out.reshape(_H, bn).T.reshape(b, n, _H)
